# Initial kernel scaffold; baseline (speedup 1.0000x reference)
import jax
import jax.numpy as jnp
from jax.experimental import pallas as pl


def kernel(x, edge_index, batch, Wl1, bl1, Wr1, br1, att1, bias1, Wl2, bl2, Wr2, br2, att2, bias2):
    raise NotImplementedError("write your pallas kernel here")


# speedup vs baseline: 18.1097x; 18.1097x over previous
import functools

import jax
import jax.numpy as jnp
from jax import lax
from jax.experimental import pallas as pl
from jax.experimental.pallas import tpu as pltpu
from jax.experimental.pallas import tpu_sc as plsc

N = 10000
E = 320000
IN = 128
HID = 64
OUT = 2

NC = 2
NS = 16
NW = NC * NS
PER_W = E // NW
B1 = 400
G1 = B1 // 16
NCH1 = PER_W // B1
B2 = 2000
G2 = B2 // 16
NCH2 = PER_W // B2
ROWS_PER_CP = N // 10


def _l1_edges(xl_hbm, xr_hbm, src_hbm, dst_hbm, attv_hbm, zero80_hbm,
              numer_out,
              src_v, dst_v, rows_l, rows_r, stage, att_v, numer_s,
              sem1, sem2):
  c = lax.axis_index("c")
  s = lax.axis_index("s")
  wid = c * NS + s

  @pl.when(s < 10)
  def _():
    pltpu.sync_copy(zero80_hbm.at[pl.ds(s * ROWS_PER_CP, ROWS_PER_CP)],
                    numer_s.at[pl.ds(s * ROWS_PER_CP, ROWS_PER_CP)])

  pltpu.sync_copy(attv_hbm, att_v)
  plsc.subcore_barrier()

  a_vecs = [att_v[k, :] for k in range(4)]

  def chunk(i, carry):
    base = wid * PER_W + i * B1
    pltpu.sync_copy(src_hbm.at[pl.ds(base, B1)], src_v)
    pltpu.sync_copy(dst_hbm.at[pl.ds(base, B1)], dst_v)
    cp1 = pltpu.async_copy(xl_hbm.at[src_v], rows_l, sem1)
    cp2 = pltpu.async_copy(xr_hbm.at[dst_v], rows_r, sem2)
    cp1.wait()
    cp2.wait()

    def group(g, _):
      for j in range(16):
        e = g * 16 + j
        acc = jnp.zeros((16,), jnp.float32)
        for k in range(4):
          vl = rows_l[e, pl.ds(k * 16, 16)]
          vr = rows_r[e, pl.ds(k * 16, 16)]
          t = vl + vr
          t = jnp.maximum(t, t * 0.2)
          acc = acc + a_vecs[k] * t
        wv = jnp.exp(jnp.broadcast_to(jnp.sum(acc), (16,)))
        stage[e, pl.ds(52, 16)] = wv
        for k in range(4):
          stage[e, pl.ds(k * 16, 16)] = rows_l[e, pl.ds(k * 16, 16)] * wv
      return 0

    lax.fori_loop(0, G1, group, 0)
    pltpu.sync_copy(stage, numer_s.at[dst_v], add=True)
    return carry

  lax.fori_loop(0, NCH1, chunk, 0)
  plsc.subcore_barrier()

  @pl.when(s < 10)
  def _():
    pltpu.sync_copy(numer_s.at[pl.ds(s * ROWS_PER_CP, ROWS_PER_CP)],
                    numer_out.at[pl.ds(c * N + s * ROWS_PER_CP, ROWS_PER_CP)])


def _l2_edges(tab_hbm, src_hbm, dst_hbm, att2v_hbm, zero4_hbm, acc_out,
              tab_v, src_v, dst_v, rows2, att2_v, acc_s, sem1):
  c = lax.axis_index("c")
  s = lax.axis_index("s")
  wid = c * NS + s

  @pl.when(s < 10)
  def _():
    pltpu.sync_copy(zero4_hbm.at[pl.ds(s * ROWS_PER_CP, ROWS_PER_CP)],
                    acc_s.at[pl.ds(s * ROWS_PER_CP, ROWS_PER_CP)])

  pltpu.sync_copy(tab_hbm, tab_v)
  pltpu.sync_copy(att2v_hbm, att2_v)
  plsc.subcore_barrier()

  lane = jnp.arange(16, dtype=jnp.int32)
  i0 = jnp.zeros((16,), jnp.int32)
  i1 = i0 + 1
  i2 = i0 + 2
  i3 = i0 + 3
  a0 = att2_v[0, :]
  a1 = att2_v[1, :]
  zf = jnp.zeros((16,), jnp.float32)

  def chunk(i, carry):
    base = wid * PER_W + i * B2
    pltpu.sync_copy(src_hbm.at[pl.ds(base, B2)], src_v)
    pltpu.sync_copy(dst_hbm.at[pl.ds(base, B2)], dst_v)

    def group(g, _):
      sv = src_v[pl.ds(g * 16, 16)]
      dv = dst_v[pl.ds(g * 16, 16)]
      l0 = plsc.load_gather(tab_v, [sv, i0])
      l1 = plsc.load_gather(tab_v, [sv, i1])
      r0 = plsc.load_gather(tab_v, [dv, i2])
      r1 = plsc.load_gather(tab_v, [dv, i3])
      t0 = l0 + r0
      t0 = jnp.maximum(t0, t0 * 0.2)
      t1 = l1 + r1
      t1 = jnp.maximum(t1, t1 * 0.2)
      w = jnp.exp(a0 * t0 + a1 * t1)
      eidx = g * 16 + lane
      plsc.store_scatter(rows2, [eidx, i0], w * l0)
      plsc.store_scatter(rows2, [eidx, i1], w * l1)
      plsc.store_scatter(rows2, [eidx, i2], w)
      plsc.store_scatter(rows2, [eidx, i3], zf)
      return 0

    lax.fori_loop(0, G2, group, 0)
    pltpu.sync_copy(rows2, acc_s.at[dst_v], add=True)
    return carry

  lax.fori_loop(0, NCH2, chunk, 0)
  plsc.subcore_barrier()

  @pl.when(s < 10)
  def _():
    pltpu.sync_copy(acc_s.at[pl.ds(s * ROWS_PER_CP, ROWS_PER_CP)],
                    acc_out.at[pl.ds(c * N + s * ROWS_PER_CP, ROWS_PER_CP)])


def _tc_in_body(x_ref, w_ref, b_ref, xl_ref, xr_ref):
  y = jnp.dot(x_ref[...], w_ref[...], preferred_element_type=jnp.float32)
  y = y + b_ref[...]
  xl_ref[...] = y[:, :HID]
  xr_ref[...] = y[:, HID:]


def _tc_mid_body(p0_ref, p1_ref, b1_ref, w2_ref, b2_ref, o_ref):
  p = p0_ref[...] + p1_ref[...]
  h = p[:, :HID] / (p[:, HID:HID + 1] + 1e-16)
  h = h + b1_ref[...]
  h = jnp.where(h > 0, h, jnp.exp(jnp.minimum(h, 0.0)) - 1.0)
  o_ref[...] = (
      jnp.dot(h, w2_ref[...], preferred_element_type=jnp.float32)
      + b2_ref[...]
  )


def _tc_fin_body(a0_ref, a1_ref, b_ref, o_ref):
  a = a0_ref[...] + a1_ref[...]
  o_ref[...] = a[:, :OUT] / (a[:, OUT:OUT + 1] + 1e-16) + b_ref[...]


_ROWBLK = 2000


def _tc_in(x, wcat_t, bcat):
  return pl.pallas_call(
      _tc_in_body,
      grid=(N // _ROWBLK,),
      in_specs=[
          pl.BlockSpec((_ROWBLK, IN), lambda i: (i, 0)),
          pl.BlockSpec((IN, 2 * HID), lambda i: (0, 0)),
          pl.BlockSpec((1, 2 * HID), lambda i: (0, 0)),
      ],
      out_specs=[
          pl.BlockSpec((_ROWBLK, HID), lambda i: (i, 0)),
          pl.BlockSpec((_ROWBLK, HID), lambda i: (i, 0)),
      ],
      out_shape=[
          jax.ShapeDtypeStruct((N, HID), jnp.float32),
          jax.ShapeDtypeStruct((N, HID), jnp.float32),
      ],
  )(x, wcat_t, bcat)


def _tc_mid(p0, p1, b1, w2t, b2):
  return pl.pallas_call(
      _tc_mid_body,
      grid=(N // _ROWBLK,),
      in_specs=[
          pl.BlockSpec((_ROWBLK, 68), lambda i: (i, 0)),
          pl.BlockSpec((_ROWBLK, 68), lambda i: (i, 0)),
          pl.BlockSpec((1, HID), lambda i: (0, 0)),
          pl.BlockSpec((HID, 4), lambda i: (0, 0)),
          pl.BlockSpec((1, 4), lambda i: (0, 0)),
      ],
      out_specs=pl.BlockSpec((_ROWBLK, 4), lambda i: (i, 0)),
      out_shape=jax.ShapeDtypeStruct((N, 4), jnp.float32),
  )(p0, p1, b1, w2t, b2)


def _tc_fin(a0, a1, b2):
  return pl.pallas_call(
      _tc_fin_body,
      grid=(N // _ROWBLK,),
      in_specs=[
          pl.BlockSpec((_ROWBLK, 4), lambda i: (i, 0)),
          pl.BlockSpec((_ROWBLK, 4), lambda i: (i, 0)),
          pl.BlockSpec((1, OUT), lambda i: (0, 0)),
      ],
      out_specs=pl.BlockSpec((_ROWBLK, OUT), lambda i: (i, 0)),
      out_shape=jax.ShapeDtypeStruct((N, OUT), jnp.float32),
  )(a0, a1, b2)


def kernel(x, edge_index, batch, Wl1, bl1, Wr1, br1, att1, bias1,
           Wl2, bl2, Wr2, br2, att2, bias2):
  del batch
  src = edge_index[0]
  dst = edge_index[1]

  wcat_t = jnp.concatenate([Wl1, Wr1], axis=0).T
  bcat = jnp.concatenate([bl1, br1], axis=0)[None, :]
  xl, xr = _tc_in(x, wcat_t, bcat)

  attv = att1.reshape(4, 16)
  zero80 = jnp.zeros((N, 68), jnp.float32)

  mesh = plsc.VectorSubcoreMesh(core_axis_name="c", subcore_axis_name="s",
                                num_cores=NC, num_subcores=NS)
  sc_params = pltpu.CompilerParams(needs_layout_passes=False,
                                   use_tc_tiling_on_sc=False)
  l1 = pl.kernel(
      _l1_edges,
      compiler_params=sc_params,
      out_type=jax.ShapeDtypeStruct((NC * N, 68), jnp.float32),
      mesh=mesh,
      scratch_types=[
          pltpu.VMEM((B1,), jnp.int32),
          pltpu.VMEM((B1,), jnp.int32),
          pltpu.VMEM((B1, HID), jnp.float32),
          pltpu.VMEM((B1, HID), jnp.float32),
          pltpu.VMEM((B1, 68), jnp.float32),
          pltpu.VMEM((4, 16), jnp.float32),
          pltpu.MemorySpace.VMEM_SHARED((N, 68), jnp.float32),
          pltpu.SemaphoreType.DMA,
          pltpu.SemaphoreType.DMA,
      ],
  )
  numer = l1(xl, xr, src, dst, attv, zero80)

  w2t = jnp.concatenate([Wl2, Wr2], axis=0).T
  b2 = jnp.concatenate([bl2, br2], axis=0)[None, :]
  tab2 = _tc_mid(numer[:N], numer[N:], bias1[None, :], w2t, b2)

  att2v = jnp.broadcast_to(att2.reshape(2, 1), (2, 16))
  zero4 = jnp.zeros((N, 4), jnp.float32)

  l2 = pl.kernel(
      _l2_edges,
      compiler_params=sc_params,
      out_type=jax.ShapeDtypeStruct((NC * N, 4), jnp.float32),
      mesh=mesh,
      scratch_types=[
          pltpu.VMEM((N, 4), jnp.float32),
          pltpu.VMEM((B2,), jnp.int32),
          pltpu.VMEM((B2,), jnp.int32),
          pltpu.VMEM((B2, 4), jnp.float32),
          pltpu.VMEM((2, 16), jnp.float32),
          pltpu.MemorySpace.VMEM_SHARED((N, 4), jnp.float32),
          pltpu.SemaphoreType.DMA,
      ],
  )
  acc = l2(tab2, src, dst, att2v, zero4)

  return _tc_fin(acc[:N], acc[N:], bias2[None, :])


# speedup vs baseline: 20.1539x; 1.1129x over previous
import functools

import jax
import jax.numpy as jnp
from jax import lax
from jax.experimental import pallas as pl
from jax.experimental.pallas import tpu as pltpu
from jax.experimental.pallas import tpu_sc as plsc

N = 10000
E = 320000
IN = 128
HID = 64
OUT = 2

NC = 2
NS = 16
NW = NC * NS
PER_W = E // NW
B1 = 400
G1 = B1 // 16
NCH1 = PER_W // B1
B2 = 2000
G2 = B2 // 16
NCH2 = PER_W // B2
ROWS_PER_CP = N // 10


def _l1_edges(xl_hbm, xr_hbm, src_hbm, dst_hbm, attv_hbm, zero80_hbm,
              numer_out,
              src_v, dst_v, rows_l, rows_r, stage, att_v, acc_m, w_sc,
              numer_s, sem1, sem2):
  c = lax.axis_index("c")
  s = lax.axis_index("s")
  wid = c * NS + s

  @pl.when(s < 10)
  def _():
    pltpu.sync_copy(zero80_hbm.at[pl.ds(s * ROWS_PER_CP, ROWS_PER_CP)],
                    numer_s.at[pl.ds(s * ROWS_PER_CP, ROWS_PER_CP)])

  pltpu.sync_copy(attv_hbm, att_v)
  plsc.subcore_barrier()

  a_vecs = [att_v[k, :] for k in range(4)]

  def chunk(i, carry):
    base = wid * PER_W + i * B1
    pltpu.sync_copy(src_hbm.at[pl.ds(base, B1)], src_v)
    pltpu.sync_copy(dst_hbm.at[pl.ds(base, B1)], dst_v)
    cp1 = pltpu.async_copy(xl_hbm.at[src_v], rows_l, sem1)
    cp2 = pltpu.async_copy(xr_hbm.at[dst_v], rows_r, sem2)
    cp1.wait()
    cp2.wait()

    lane = jnp.arange(16, dtype=jnp.int32)

    def group(g, _):
      for j in range(16):
        e = g * 16 + j
        acc = jnp.zeros((16,), jnp.float32)
        for k in range(4):
          vl = rows_l[e, pl.ds(k * 16, 16)]
          vr = rows_r[e, pl.ds(k * 16, 16)]
          t = vl + vr
          t = jnp.maximum(t, t * 0.2)
          acc = acc + a_vecs[k] * t
        acc_m[j, :] = acc
      S = jnp.zeros((16,), jnp.float32)
      for k in range(16):
        S = S + plsc.load_gather(acc_m, [lane, jnp.full((16,), k, jnp.int32)])
      w_sc[...] = jnp.exp(S)
      for j in range(16):
        e = g * 16 + j
        wv = plsc.load_gather(w_sc, [jnp.full((16,), j, jnp.int32)])
        stage[e, pl.ds(52, 16)] = wv
        for k in range(4):
          stage[e, pl.ds(k * 16, 16)] = rows_l[e, pl.ds(k * 16, 16)] * wv
      return 0

    lax.fori_loop(0, G1, group, 0)
    pltpu.sync_copy(stage, numer_s.at[dst_v], add=True)
    return carry

  lax.fori_loop(0, NCH1, chunk, 0)
  plsc.subcore_barrier()

  @pl.when(s < 10)
  def _():
    pltpu.sync_copy(numer_s.at[pl.ds(s * ROWS_PER_CP, ROWS_PER_CP)],
                    numer_out.at[pl.ds(c * N + s * ROWS_PER_CP, ROWS_PER_CP)])


def _l2_edges(tab_hbm, src_hbm, dst_hbm, att2v_hbm, zero4_hbm, acc_out,
              tab_v, src_v, dst_v, rows2, att2_v, acc_s, sem1):
  c = lax.axis_index("c")
  s = lax.axis_index("s")
  wid = c * NS + s

  @pl.when(s < 10)
  def _():
    pltpu.sync_copy(zero4_hbm.at[pl.ds(s * ROWS_PER_CP, ROWS_PER_CP)],
                    acc_s.at[pl.ds(s * ROWS_PER_CP, ROWS_PER_CP)])

  pltpu.sync_copy(tab_hbm, tab_v)
  pltpu.sync_copy(att2v_hbm, att2_v)
  plsc.subcore_barrier()

  lane = jnp.arange(16, dtype=jnp.int32)
  i0 = jnp.zeros((16,), jnp.int32)
  i1 = i0 + 1
  i2 = i0 + 2
  i3 = i0 + 3
  a0 = att2_v[0, :]
  a1 = att2_v[1, :]
  zf = jnp.zeros((16,), jnp.float32)

  def chunk(i, carry):
    base = wid * PER_W + i * B2
    pltpu.sync_copy(src_hbm.at[pl.ds(base, B2)], src_v)
    pltpu.sync_copy(dst_hbm.at[pl.ds(base, B2)], dst_v)

    def group(g, _):
      sv = src_v[pl.ds(g * 16, 16)]
      dv = dst_v[pl.ds(g * 16, 16)]
      l0 = plsc.load_gather(tab_v, [sv, i0])
      l1 = plsc.load_gather(tab_v, [sv, i1])
      r0 = plsc.load_gather(tab_v, [dv, i2])
      r1 = plsc.load_gather(tab_v, [dv, i3])
      t0 = l0 + r0
      t0 = jnp.maximum(t0, t0 * 0.2)
      t1 = l1 + r1
      t1 = jnp.maximum(t1, t1 * 0.2)
      w = jnp.exp(a0 * t0 + a1 * t1)
      eidx = g * 16 + lane
      plsc.store_scatter(rows2, [eidx, i0], w * l0)
      plsc.store_scatter(rows2, [eidx, i1], w * l1)
      plsc.store_scatter(rows2, [eidx, i2], w)
      plsc.store_scatter(rows2, [eidx, i3], zf)
      return 0

    lax.fori_loop(0, G2, group, 0)
    pltpu.sync_copy(rows2, acc_s.at[dst_v], add=True)
    return carry

  lax.fori_loop(0, NCH2, chunk, 0)
  plsc.subcore_barrier()

  @pl.when(s < 10)
  def _():
    pltpu.sync_copy(acc_s.at[pl.ds(s * ROWS_PER_CP, ROWS_PER_CP)],
                    acc_out.at[pl.ds(c * N + s * ROWS_PER_CP, ROWS_PER_CP)])


def _tc_in_body(x_ref, w_ref, b_ref, xl_ref, xr_ref):
  y = jnp.dot(x_ref[...], w_ref[...], preferred_element_type=jnp.float32)
  y = y + b_ref[...]
  xl_ref[...] = y[:, :HID]
  xr_ref[...] = y[:, HID:]


def _tc_mid_body(p0_ref, p1_ref, b1_ref, w2_ref, b2_ref, o_ref):
  p = p0_ref[...] + p1_ref[...]
  h = p[:, :HID] / (p[:, HID:HID + 1] + 1e-16)
  h = h + b1_ref[...]
  h = jnp.where(h > 0, h, jnp.exp(jnp.minimum(h, 0.0)) - 1.0)
  o_ref[...] = (
      jnp.dot(h, w2_ref[...], preferred_element_type=jnp.float32)
      + b2_ref[...]
  )


def _tc_fin_body(a0_ref, a1_ref, b_ref, o_ref):
  a = a0_ref[...] + a1_ref[...]
  o_ref[...] = a[:, :OUT] / (a[:, OUT:OUT + 1] + 1e-16) + b_ref[...]


_ROWBLK = 2000


def _tc_in(x, wcat_t, bcat):
  return pl.pallas_call(
      _tc_in_body,
      grid=(N // _ROWBLK,),
      in_specs=[
          pl.BlockSpec((_ROWBLK, IN), lambda i: (i, 0)),
          pl.BlockSpec((IN, 2 * HID), lambda i: (0, 0)),
          pl.BlockSpec((1, 2 * HID), lambda i: (0, 0)),
      ],
      out_specs=[
          pl.BlockSpec((_ROWBLK, HID), lambda i: (i, 0)),
          pl.BlockSpec((_ROWBLK, HID), lambda i: (i, 0)),
      ],
      out_shape=[
          jax.ShapeDtypeStruct((N, HID), jnp.float32),
          jax.ShapeDtypeStruct((N, HID), jnp.float32),
      ],
  )(x, wcat_t, bcat)


def _tc_mid(p0, p1, b1, w2t, b2):
  return pl.pallas_call(
      _tc_mid_body,
      grid=(N // _ROWBLK,),
      in_specs=[
          pl.BlockSpec((_ROWBLK, 68), lambda i: (i, 0)),
          pl.BlockSpec((_ROWBLK, 68), lambda i: (i, 0)),
          pl.BlockSpec((1, HID), lambda i: (0, 0)),
          pl.BlockSpec((HID, 4), lambda i: (0, 0)),
          pl.BlockSpec((1, 4), lambda i: (0, 0)),
      ],
      out_specs=pl.BlockSpec((_ROWBLK, 4), lambda i: (i, 0)),
      out_shape=jax.ShapeDtypeStruct((N, 4), jnp.float32),
  )(p0, p1, b1, w2t, b2)


def _tc_fin(a0, a1, b2):
  return pl.pallas_call(
      _tc_fin_body,
      grid=(N // _ROWBLK,),
      in_specs=[
          pl.BlockSpec((_ROWBLK, 4), lambda i: (i, 0)),
          pl.BlockSpec((_ROWBLK, 4), lambda i: (i, 0)),
          pl.BlockSpec((1, OUT), lambda i: (0, 0)),
      ],
      out_specs=pl.BlockSpec((_ROWBLK, OUT), lambda i: (i, 0)),
      out_shape=jax.ShapeDtypeStruct((N, OUT), jnp.float32),
  )(a0, a1, b2)


def kernel(x, edge_index, batch, Wl1, bl1, Wr1, br1, att1, bias1,
           Wl2, bl2, Wr2, br2, att2, bias2):
  del batch
  src = edge_index[0]
  dst = edge_index[1]

  wcat_t = jnp.concatenate([Wl1, Wr1], axis=0).T
  bcat = jnp.concatenate([bl1, br1], axis=0)[None, :]
  xl, xr = _tc_in(x, wcat_t, bcat)

  attv = att1.reshape(4, 16)
  zero80 = jnp.zeros((N, 68), jnp.float32)

  mesh = plsc.VectorSubcoreMesh(core_axis_name="c", subcore_axis_name="s",
                                num_cores=NC, num_subcores=NS)
  sc_params = pltpu.CompilerParams(needs_layout_passes=False,
                                   use_tc_tiling_on_sc=False)
  l1 = pl.kernel(
      _l1_edges,
      compiler_params=sc_params,
      out_type=jax.ShapeDtypeStruct((NC * N, 68), jnp.float32),
      mesh=mesh,
      scratch_types=[
          pltpu.VMEM((B1,), jnp.int32),
          pltpu.VMEM((B1,), jnp.int32),
          pltpu.VMEM((B1, HID), jnp.float32),
          pltpu.VMEM((B1, HID), jnp.float32),
          pltpu.VMEM((B1, 68), jnp.float32),
          pltpu.VMEM((4, 16), jnp.float32),
          pltpu.VMEM((16, 16), jnp.float32),
          pltpu.VMEM((16,), jnp.float32),
          pltpu.MemorySpace.VMEM_SHARED((N, 68), jnp.float32),
          pltpu.SemaphoreType.DMA,
          pltpu.SemaphoreType.DMA,
      ],
  )
  numer = l1(xl, xr, src, dst, attv, zero80)

  w2t = jnp.concatenate([Wl2, Wr2], axis=0).T
  b2 = jnp.concatenate([bl2, br2], axis=0)[None, :]
  tab2 = _tc_mid(numer[:N], numer[N:], bias1[None, :], w2t, b2)

  att2v = jnp.broadcast_to(att2.reshape(2, 1), (2, 16))
  zero4 = jnp.zeros((N, 4), jnp.float32)

  l2 = pl.kernel(
      _l2_edges,
      compiler_params=sc_params,
      out_type=jax.ShapeDtypeStruct((NC * N, 4), jnp.float32),
      mesh=mesh,
      scratch_types=[
          pltpu.VMEM((N, 4), jnp.float32),
          pltpu.VMEM((B2,), jnp.int32),
          pltpu.VMEM((B2,), jnp.int32),
          pltpu.VMEM((B2, 4), jnp.float32),
          pltpu.VMEM((2, 16), jnp.float32),
          pltpu.MemorySpace.VMEM_SHARED((N, 4), jnp.float32),
          pltpu.SemaphoreType.DMA,
      ],
  )
  acc = l2(tab2, src, dst, att2v, zero4)

  return _tc_fin(acc[:N], acc[N:], bias2[None, :])


# speedup vs baseline: 20.2213x; 1.0033x over previous
import functools

import jax
import jax.numpy as jnp
from jax import lax
from jax.experimental import pallas as pl
from jax.experimental.pallas import tpu as pltpu
from jax.experimental.pallas import tpu_sc as plsc

N = 10000
E = 320000
IN = 128
HID = 64
OUT = 2

NC = 2
NS = 16
NW = NC * NS
PER_W = E // NW
B1 = 400
G1 = B1 // 16
NCH1 = PER_W // B1
B2 = 2000
G2 = B2 // 16
NCH2 = PER_W // B2
ROWS_PER_CP = N // 10


def _l1_edges(xl_hbm, xr_hbm, src_hbm, dst_hbm, attv_hbm, zero80_hbm,
              numer_out,
              src_v, dst_v, rows_l, rows_r, stage, att_v, acc_m, w_sc,
              numer_s, sem1, sem2):
  c = lax.axis_index("c")
  s = lax.axis_index("s")
  wid = c * NS + s

  @pl.when(s < 10)
  def _():
    pltpu.sync_copy(zero80_hbm.at[pl.ds(s * ROWS_PER_CP, ROWS_PER_CP)],
                    numer_s.at[pl.ds(s * ROWS_PER_CP, ROWS_PER_CP)])

  pltpu.sync_copy(attv_hbm, att_v)
  plsc.subcore_barrier()

  a_vecs = [att_v[k, :] for k in range(4)]

  def chunk(i, carry):
    base = wid * PER_W + i * B1
    pltpu.sync_copy(src_hbm.at[pl.ds(base, B1)], src_v)
    pltpu.sync_copy(dst_hbm.at[pl.ds(base, B1)], dst_v)
    cp1 = pltpu.async_copy(xl_hbm.at[src_v], rows_l, sem1)
    cp2 = pltpu.async_copy(xr_hbm.at[dst_v], rows_r, sem2)
    cp1.wait()
    cp2.wait()

    lane = jnp.arange(16, dtype=jnp.int32)

    def group(g, _):
      for j in range(16):
        e = g * 16 + j
        acc = jnp.zeros((16,), jnp.float32)
        for k in range(4):
          vl = rows_l[e, pl.ds(k * 16, 16)]
          vr = rows_r[e, pl.ds(k * 16, 16)]
          t = vl + vr
          t = jnp.maximum(t, t * 0.2)
          acc = acc + a_vecs[k] * t
        acc_m[j, :] = acc
      S = jnp.zeros((16,), jnp.float32)
      for k in range(16):
        S = S + plsc.load_gather(acc_m, [lane, jnp.full((16,), k, jnp.int32)])
      w_sc[...] = jnp.exp(S)
      for j in range(16):
        e = g * 16 + j
        wv = plsc.load_gather(w_sc, [jnp.full((16,), j, jnp.int32)])
        stage[e, pl.ds(52, 16)] = wv
        for k in range(4):
          stage[e, pl.ds(k * 16, 16)] = rows_l[e, pl.ds(k * 16, 16)] * wv
      return 0

    lax.fori_loop(0, G1, group, 0, unroll=2)
    pltpu.sync_copy(stage, numer_s.at[dst_v], add=True)
    return carry

  lax.fori_loop(0, NCH1, chunk, 0)
  plsc.subcore_barrier()

  @pl.when(s < 10)
  def _():
    pltpu.sync_copy(numer_s.at[pl.ds(s * ROWS_PER_CP, ROWS_PER_CP)],
                    numer_out.at[pl.ds(c * N + s * ROWS_PER_CP, ROWS_PER_CP)])


def _l2_edges(tab_hbm, src_hbm, dst_hbm, att2v_hbm, zero4_hbm, acc_out,
              tab_v, src_v, dst_v, rows2, att2_v, acc_s, sem1):
  c = lax.axis_index("c")
  s = lax.axis_index("s")
  wid = c * NS + s

  @pl.when(s < 10)
  def _():
    pltpu.sync_copy(zero4_hbm.at[pl.ds(s * ROWS_PER_CP, ROWS_PER_CP)],
                    acc_s.at[pl.ds(s * ROWS_PER_CP, ROWS_PER_CP)])

  pltpu.sync_copy(tab_hbm, tab_v)
  pltpu.sync_copy(att2v_hbm, att2_v)
  plsc.subcore_barrier()

  lane = jnp.arange(16, dtype=jnp.int32)
  i0 = jnp.zeros((16,), jnp.int32)
  i1 = i0 + 1
  i2 = i0 + 2
  i3 = i0 + 3
  a0 = att2_v[0, :]
  a1 = att2_v[1, :]
  zf = jnp.zeros((16,), jnp.float32)

  def chunk(i, carry):
    base = wid * PER_W + i * B2
    pltpu.sync_copy(src_hbm.at[pl.ds(base, B2)], src_v)
    pltpu.sync_copy(dst_hbm.at[pl.ds(base, B2)], dst_v)

    def group(g, _):
      sv = src_v[pl.ds(g * 16, 16)]
      dv = dst_v[pl.ds(g * 16, 16)]
      l0 = plsc.load_gather(tab_v, [sv, i0])
      l1 = plsc.load_gather(tab_v, [sv, i1])
      r0 = plsc.load_gather(tab_v, [dv, i2])
      r1 = plsc.load_gather(tab_v, [dv, i3])
      t0 = l0 + r0
      t0 = jnp.maximum(t0, t0 * 0.2)
      t1 = l1 + r1
      t1 = jnp.maximum(t1, t1 * 0.2)
      w = jnp.exp(a0 * t0 + a1 * t1)
      eidx = g * 16 + lane
      plsc.store_scatter(rows2, [eidx, i0], w * l0)
      plsc.store_scatter(rows2, [eidx, i1], w * l1)
      plsc.store_scatter(rows2, [eidx, i2], w)
      plsc.store_scatter(rows2, [eidx, i3], zf)
      return 0

    lax.fori_loop(0, G2, group, 0)
    pltpu.sync_copy(rows2, acc_s.at[dst_v], add=True)
    return carry

  lax.fori_loop(0, NCH2, chunk, 0)
  plsc.subcore_barrier()

  @pl.when(s < 10)
  def _():
    pltpu.sync_copy(acc_s.at[pl.ds(s * ROWS_PER_CP, ROWS_PER_CP)],
                    acc_out.at[pl.ds(c * N + s * ROWS_PER_CP, ROWS_PER_CP)])


def _tc_in_body(x_ref, w_ref, b_ref, xl_ref, xr_ref):
  y = jnp.dot(x_ref[...], w_ref[...], preferred_element_type=jnp.float32)
  y = y + b_ref[...]
  xl_ref[...] = y[:, :HID]
  xr_ref[...] = y[:, HID:]


def _tc_mid_body(p0_ref, p1_ref, b1_ref, w2_ref, b2_ref, o_ref):
  p = p0_ref[...] + p1_ref[...]
  h = p[:, :HID] / (p[:, HID:HID + 1] + 1e-16)
  h = h + b1_ref[...]
  h = jnp.where(h > 0, h, jnp.exp(jnp.minimum(h, 0.0)) - 1.0)
  o_ref[...] = (
      jnp.dot(h, w2_ref[...], preferred_element_type=jnp.float32)
      + b2_ref[...]
  )


def _tc_fin_body(a0_ref, a1_ref, b_ref, o_ref):
  a = a0_ref[...] + a1_ref[...]
  o_ref[...] = a[:, :OUT] / (a[:, OUT:OUT + 1] + 1e-16) + b_ref[...]


_ROWBLK = 2000


def _tc_in(x, wcat_t, bcat):
  return pl.pallas_call(
      _tc_in_body,
      grid=(N // _ROWBLK,),
      in_specs=[
          pl.BlockSpec((_ROWBLK, IN), lambda i: (i, 0)),
          pl.BlockSpec((IN, 2 * HID), lambda i: (0, 0)),
          pl.BlockSpec((1, 2 * HID), lambda i: (0, 0)),
      ],
      out_specs=[
          pl.BlockSpec((_ROWBLK, HID), lambda i: (i, 0)),
          pl.BlockSpec((_ROWBLK, HID), lambda i: (i, 0)),
      ],
      out_shape=[
          jax.ShapeDtypeStruct((N, HID), jnp.float32),
          jax.ShapeDtypeStruct((N, HID), jnp.float32),
      ],
  )(x, wcat_t, bcat)


def _tc_mid(p0, p1, b1, w2t, b2):
  return pl.pallas_call(
      _tc_mid_body,
      grid=(N // _ROWBLK,),
      in_specs=[
          pl.BlockSpec((_ROWBLK, 68), lambda i: (i, 0)),
          pl.BlockSpec((_ROWBLK, 68), lambda i: (i, 0)),
          pl.BlockSpec((1, HID), lambda i: (0, 0)),
          pl.BlockSpec((HID, 4), lambda i: (0, 0)),
          pl.BlockSpec((1, 4), lambda i: (0, 0)),
      ],
      out_specs=pl.BlockSpec((_ROWBLK, 4), lambda i: (i, 0)),
      out_shape=jax.ShapeDtypeStruct((N, 4), jnp.float32),
  )(p0, p1, b1, w2t, b2)


def _tc_fin(a0, a1, b2):
  return pl.pallas_call(
      _tc_fin_body,
      grid=(N // _ROWBLK,),
      in_specs=[
          pl.BlockSpec((_ROWBLK, 4), lambda i: (i, 0)),
          pl.BlockSpec((_ROWBLK, 4), lambda i: (i, 0)),
          pl.BlockSpec((1, OUT), lambda i: (0, 0)),
      ],
      out_specs=pl.BlockSpec((_ROWBLK, OUT), lambda i: (i, 0)),
      out_shape=jax.ShapeDtypeStruct((N, OUT), jnp.float32),
  )(a0, a1, b2)


def kernel(x, edge_index, batch, Wl1, bl1, Wr1, br1, att1, bias1,
           Wl2, bl2, Wr2, br2, att2, bias2):
  del batch
  src = edge_index[0]
  dst = edge_index[1]

  wcat_t = jnp.concatenate([Wl1, Wr1], axis=0).T
  bcat = jnp.concatenate([bl1, br1], axis=0)[None, :]
  xl, xr = _tc_in(x, wcat_t, bcat)

  attv = att1.reshape(4, 16)
  zero80 = jnp.zeros((N, 68), jnp.float32)

  mesh = plsc.VectorSubcoreMesh(core_axis_name="c", subcore_axis_name="s",
                                num_cores=NC, num_subcores=NS)
  sc_params = pltpu.CompilerParams(needs_layout_passes=False,
                                   use_tc_tiling_on_sc=False)
  l1 = pl.kernel(
      _l1_edges,
      compiler_params=sc_params,
      out_type=jax.ShapeDtypeStruct((NC * N, 68), jnp.float32),
      mesh=mesh,
      scratch_types=[
          pltpu.VMEM((B1,), jnp.int32),
          pltpu.VMEM((B1,), jnp.int32),
          pltpu.VMEM((B1, HID), jnp.float32),
          pltpu.VMEM((B1, HID), jnp.float32),
          pltpu.VMEM((B1, 68), jnp.float32),
          pltpu.VMEM((4, 16), jnp.float32),
          pltpu.VMEM((16, 16), jnp.float32),
          pltpu.VMEM((16,), jnp.float32),
          pltpu.MemorySpace.VMEM_SHARED((N, 68), jnp.float32),
          pltpu.SemaphoreType.DMA,
          pltpu.SemaphoreType.DMA,
      ],
  )
  numer = l1(xl, xr, src, dst, attv, zero80)

  w2t = jnp.concatenate([Wl2, Wr2], axis=0).T
  b2 = jnp.concatenate([bl2, br2], axis=0)[None, :]
  tab2 = _tc_mid(numer[:N], numer[N:], bias1[None, :], w2t, b2)

  att2v = jnp.broadcast_to(att2.reshape(2, 1), (2, 16))
  zero4 = jnp.zeros((N, 4), jnp.float32)

  l2 = pl.kernel(
      _l2_edges,
      compiler_params=sc_params,
      out_type=jax.ShapeDtypeStruct((NC * N, 4), jnp.float32),
      mesh=mesh,
      scratch_types=[
          pltpu.VMEM((N, 4), jnp.float32),
          pltpu.VMEM((B2,), jnp.int32),
          pltpu.VMEM((B2,), jnp.int32),
          pltpu.VMEM((B2, 4), jnp.float32),
          pltpu.VMEM((2, 16), jnp.float32),
          pltpu.MemorySpace.VMEM_SHARED((N, 4), jnp.float32),
          pltpu.SemaphoreType.DMA,
      ],
  )
  acc = l2(tab2, src, dst, att2v, zero4)

  return _tc_fin(acc[:N], acc[N:], bias2[None, :])
